# baseline (speedup 1.0000x reference)
import functools

import jax
import jax.numpy as jnp
from jax import lax
from jax.experimental import pallas as pl
from jax.experimental.pallas import tpu as pltpu
from jax.experimental.pallas import tpu_sc as plsc

_info = plsc.get_sparse_core_info()
_NC, _NS = _info.num_cores, _info.num_subcores
_NW = _NC * _NS

_CHUNK_BATCHES = 4


@functools.partial(jax.jit, static_argnums=(2, 3, 4))
def _embed(Wp, xf, B, L, d):
    n = B * L
    b_per_w = B // _NW
    rows_per_w = b_per_w * L
    n_chunks = b_per_w // _CHUNK_BATCHES
    chunk_rows = _CHUNK_BATCHES * L
    mesh = plsc.VectorSubcoreMesh(core_axis_name="c", subcore_axis_name="s")

    @functools.partial(
        pl.kernel,
        mesh=mesh,
        out_type=jax.ShapeDtypeStruct((B, L, d), jnp.float32),
        scratch_types=[
            pltpu.VMEM((rows_per_w,), jnp.int32),
            pltpu.VMEM((2, chunk_rows, d), jnp.float32),
            pltpu.SemaphoreType.DMA,
            pltpu.SemaphoreType.DMA,
        ],
        compiler_params=pltpu.CompilerParams(use_tc_tiling_on_sc=False),
    )
    def k(table_hbm, idx_hbm, out_hbm, idx_v, rows_v, sem_g, sem_w):
        wid = lax.axis_index("s") * _NC + lax.axis_index("c")
        base_b = wid * b_per_w
        pltpu.sync_copy(idx_hbm.at[pl.ds(wid * rows_per_w, rows_per_w)], idx_v)

        def start_gather(j):
            return pltpu.async_copy(
                table_hbm.at[idx_v.at[pl.ds(j * chunk_rows, chunk_rows)]],
                rows_v.at[j % 2],
                sem_g,
            )

        def start_writes(j):
            return [
                pltpu.async_copy(
                    rows_v.at[j % 2, pl.ds(kk * L, L)],
                    out_hbm.at[base_b + j * _CHUNK_BATCHES + kk],
                    sem_w,
                )
                for kk in range(_CHUNK_BATCHES)
            ]

        gathers = [None] * n_chunks
        writes = [None] * n_chunks
        gathers[0] = start_gather(0)
        for j in range(n_chunks):
            if j + 1 < n_chunks:
                if j >= 1:
                    for w in writes[j - 1]:
                        w.wait()
                gathers[j + 1] = start_gather(j + 1)
            gathers[j].wait()
            writes[j] = start_writes(j)
        for j in (n_chunks - 2, n_chunks - 1):
            for w in writes[j]:
                w.wait()

    return k(Wp, xf)


def kernel(x, W):
    B, L = x.shape
    V, D = W.shape
    return _embed(W, x.reshape(B * L), B, L, D)


# speedup vs baseline: 1.0699x; 1.0699x over previous
import functools

import jax
import jax.numpy as jnp
from jax import lax
from jax.experimental import pallas as pl
from jax.experimental.pallas import tpu as pltpu
from jax.experimental.pallas import tpu_sc as plsc

_info = plsc.get_sparse_core_info()
_NC, _NS = _info.num_cores, _info.num_subcores
_NW = _NC * _NS

_CB = 4


@functools.partial(jax.jit, static_argnums=(2, 3, 4))
def _embed(Wp, xf, B, L, d):
    b_per_w = B // _NW
    rows_per_w = b_per_w * L
    n_chunks = b_per_w // _CB
    chunk_rows = _CB * L
    mesh = plsc.VectorSubcoreMesh(core_axis_name="c", subcore_axis_name="s")

    @functools.partial(
        pl.kernel,
        mesh=mesh,
        out_type=jax.ShapeDtypeStruct((B, L, d), jnp.float32),
        scratch_types=[
            pltpu.VMEM((rows_per_w,), jnp.int32),
            pltpu.VMEM((2, chunk_rows, 128), jnp.float32),
            pltpu.VMEM((2, _CB, L, d), jnp.float32),
            pltpu.SemaphoreType.DMA,
            pltpu.SemaphoreType.DMA,
        ],
    )
    def k(table_hbm, idx_hbm, out_hbm, idx_v, rows_v, tiled_v, sem_g, sem_w):
        wid = lax.axis_index("s") * _NC + lax.axis_index("c")
        base_b = wid * b_per_w
        pltpu.sync_copy(idx_hbm.at[pl.ds(wid * rows_per_w, rows_per_w)], idx_v)

        def start_gather(j):
            return pltpu.async_copy(
                table_hbm.at[idx_v.at[pl.ds(j * chunk_rows, chunk_rows)]],
                rows_v.at[j % 2],
                sem_g,
            )

        def compact(j):
            p = j % 2

            def body(t, _):
                for i in range(_CB):
                    for c in range(d // 16):
                        tiled_v[p, i, t, pl.ds(c * 16, 16)] = rows_v[
                            p, i * L + t, pl.ds(c * 16, 16)
                        ]
                return ()

            lax.fori_loop(0, L, body, (), unroll=False)

        def start_write(j):
            return pltpu.async_copy(
                tiled_v.at[j % 2],
                out_hbm.at[pl.ds(base_b + j * _CB, _CB)],
                sem_w,
            )

        gathers = [None] * n_chunks
        writes = [None] * n_chunks
        gathers[0] = start_gather(0)
        for j in range(n_chunks):
            if j + 1 < n_chunks:
                gathers[j + 1] = start_gather(j + 1)
            gathers[j].wait()
            if j >= 2:
                writes[j - 2].wait()
            compact(j)
            writes[j] = start_write(j)
        writes[n_chunks - 2].wait()
        writes[n_chunks - 1].wait()

    return k(Wp, xf)


def kernel(x, W):
    B, L = x.shape
    V, D = W.shape
    Wp = jnp.pad(W, ((0, 0), (0, 128 - D)))
    return _embed(Wp, x.reshape(B * L), B, L, D)


# speedup vs baseline: 1.2589x; 1.1766x over previous
import functools

import jax
import jax.numpy as jnp
from jax import lax
from jax.experimental import pallas as pl
from jax.experimental.pallas import tpu as pltpu
from jax.experimental.pallas import tpu_sc as plsc

_info = plsc.get_sparse_core_info()
_NC, _NS = _info.num_cores, _info.num_subcores
_NW = _NC * _NS

_DB = 8
_VP = 1024


@functools.partial(jax.jit, static_argnums=(2, 3, 4))
def _embed_t(Wt, xT, B, L, d):
    nd = d // _DB
    n_units = L * nd
    mesh = plsc.VectorSubcoreMesh(core_axis_name="c", subcore_axis_name="s")
    base = n_units // _NW
    rem = n_units % _NW
    assert base >= 2

    @functools.partial(
        pl.kernel,
        mesh=mesh,
        out_type=jax.ShapeDtypeStruct((L, d, B), jnp.float32),
        scratch_types=[
            pltpu.VMEM((d, _VP), jnp.float32),
            pltpu.VMEM((2, 1, B), jnp.int32),
            pltpu.VMEM((2, 1, _DB, B), jnp.float32),
            pltpu.SemaphoreType.DMA,
            pltpu.SemaphoreType.DMA,
            pltpu.SemaphoreType.DMA,
        ],
        compiler_params=pltpu.CompilerParams(needs_layout_passes=False),
    )
    def k(wt_hbm, xt_hbm, out_hbm, wt_v, idx_v, out_v, sem_t, sem_i, sem_o):
        wid = lax.axis_index("s") * _NC + lax.axis_index("c")
        u0 = wid * base + jnp.minimum(wid, rem)
        u1 = u0 + base + jnp.where(wid < rem, 1, 0)

        def start_idx(u, p):
            pltpu.async_copy(xt_hbm.at[pl.ds(u // nd, 1)], idx_v.at[p], sem_i)

        def wait_idx():
            pltpu.make_async_copy(xt_hbm.at[pl.ds(0, 1)], idx_v.at[0], sem_i).wait()

        def start_out(u, p):
            dt = u % nd
            pltpu.async_copy(
                out_v.at[p],
                out_hbm.at[
                    pl.ds(u // nd, 1),
                    pl.ds(pl.multiple_of(dt * _DB, _DB), _DB),
                ],
                sem_o,
            )

        def wait_out():
            pltpu.make_async_copy(
                out_v.at[0], out_hbm.at[pl.ds(0, 1), pl.ds(0, _DB)], sem_o
            ).wait()

        tbl = pltpu.async_copy(wt_hbm, wt_v, sem_t)
        start_idx(u0, 0)
        tbl.wait()

        def unit(u, _):
            p = lax.rem(u - u0, 2)
            dt = u % nd

            @pl.when(u + 1 < u1)
            def _():
                start_idx(u + 1, 1 - p)

            wait_idx()

            @pl.when(u - u0 >= 2)
            def _():
                wait_out()

            def col(g, _):
                iv = idx_v[p, 0, pl.ds(g * 16, 16)]
                for d8 in range(_DB):
                    row = jnp.full((16,), dt * _DB + d8, jnp.int32)
                    out_v[p, 0, d8, pl.ds(g * 16, 16)] = plsc.load_gather(
                        wt_v, [row, iv]
                    )
                return ()

            lax.fori_loop(0, B // 16, col, (), unroll=2)
            start_out(u, p)
            return ()

        lax.fori_loop(u0, u1, unit, ())
        wait_out()
        wait_out()

    return k(Wt, xT)


def kernel(x, W):
    B, L = x.shape
    V, D = W.shape
    Wt = jnp.pad(W.T, ((0, 0), (0, _VP - V)))
    out_t = _embed_t(Wt, x.T, B, L, D)
    return jnp.transpose(out_t, (2, 0, 1))


# speedup vs baseline: 1.2910x; 1.0255x over previous
import functools

import jax
import jax.numpy as jnp
from jax import lax
from jax.experimental import pallas as pl
from jax.experimental.pallas import tpu as pltpu
from jax.experimental.pallas import tpu_sc as plsc

_info = plsc.get_sparse_core_info()
_NC, _NS = _info.num_cores, _info.num_subcores
_NW = _NC * _NS

_DB = 8
_VP = 1024


@functools.partial(jax.jit, static_argnums=(2, 3, 4))
def _embed_t(Wt, xT, B, L, d):
    nd = d // _DB
    n_units = L * nd
    mesh = plsc.VectorSubcoreMesh(core_axis_name="c", subcore_axis_name="s")
    base = n_units // _NW
    rem = n_units % _NW
    assert base >= 2

    @functools.partial(
        pl.kernel,
        mesh=mesh,
        out_type=jax.ShapeDtypeStruct((L, d, B), jnp.float32),
        scratch_types=[
            pltpu.VMEM((d * _VP,), jnp.float32),
            pltpu.VMEM((2, 1, B), jnp.int32),
            pltpu.VMEM((2, 1, _DB, B), jnp.float32),
            pltpu.SemaphoreType.DMA,
            pltpu.SemaphoreType.DMA,
            pltpu.SemaphoreType.DMA,
        ],
        compiler_params=pltpu.CompilerParams(needs_layout_passes=False),
    )
    def k(wt_hbm, xt_hbm, out_hbm, wt_v, idx_v, out_v, sem_t, sem_i, sem_o):
        wid = lax.axis_index("s") * _NC + lax.axis_index("c")
        u0 = wid * base + jnp.minimum(wid, rem)
        u1 = u0 + base + jnp.where(wid < rem, 1, 0)

        def start_idx(u, p):
            pltpu.async_copy(xt_hbm.at[pl.ds(u // nd, 1)], idx_v.at[p], sem_i)

        def wait_idx():
            pltpu.make_async_copy(xt_hbm.at[pl.ds(0, 1)], idx_v.at[0], sem_i).wait()

        def start_out(u, p):
            dt = u % nd
            pltpu.async_copy(
                out_v.at[p],
                out_hbm.at[
                    pl.ds(u // nd, 1),
                    pl.ds(pl.multiple_of(dt * _DB, _DB), _DB),
                ],
                sem_o,
            )

        def wait_out():
            pltpu.make_async_copy(
                out_v.at[0], out_hbm.at[pl.ds(0, 1), pl.ds(0, _DB)], sem_o
            ).wait()

        tbl = pltpu.async_copy(wt_hbm, wt_v, sem_t)
        start_idx(u0, 0)
        tbl.wait()

        def unit(u, _):
            p = lax.rem(u - u0, 2)
            dt = u % nd

            @pl.when(u + 1 < u1)
            def _():
                start_idx(u + 1, 1 - p)

            wait_idx()

            @pl.when(u - u0 >= 2)
            def _():
                wait_out()

            bases = [
                jnp.full((16,), (dt * _DB + d8) * _VP, jnp.int32)
                for d8 in range(_DB)
            ]

            def col(g, _):
                iv = idx_v[p, 0, pl.ds(g * 16, 16)]
                for d8 in range(_DB):
                    out_v[p, 0, d8, pl.ds(g * 16, 16)] = plsc.load_gather(
                        wt_v, [iv + bases[d8]]
                    )
                return ()

            lax.fori_loop(0, B // 16, col, (), unroll=2)
            start_out(u, p)
            return ()

        lax.fori_loop(u0, u1, unit, ())
        wait_out()
        wait_out()

    return k(Wt, xT)


def kernel(x, W):
    B, L = x.shape
    V, D = W.shape
    Wt = jnp.pad(W.T, ((0, 0), (0, _VP - V))).reshape(-1)
    out_t = _embed_t(Wt, x.T, B, L, D)
    return jnp.transpose(out_t, (2, 0, 1))


# speedup vs baseline: 1.7571x; 1.3610x over previous
import functools

import jax
import jax.numpy as jnp
from jax import lax
from jax.experimental import pallas as pl
from jax.experimental.pallas import tpu as pltpu
from jax.experimental.pallas import tpu_sc as plsc

_info = plsc.get_sparse_core_info()
_NC, _NS = _info.num_cores, _info.num_subcores
_NW = _NC * _NS

_DB = 8
_VP = 1024


@functools.partial(jax.jit, static_argnums=(2, 3, 4))
def _embed_t(Wt, xT, B, L, d):
    nd = d // _DB
    n_units = L * nd
    mesh = plsc.VectorSubcoreMesh(core_axis_name="c", subcore_axis_name="s")
    base = n_units // _NW
    rem = n_units % _NW
    assert base >= 2

    @functools.partial(
        pl.kernel,
        mesh=mesh,
        out_type=jax.ShapeDtypeStruct((L, d, B), jnp.float32),
        scratch_types=[
            pltpu.VMEM((d * _VP,), jnp.float32),
            pltpu.VMEM((2, 1, B), jnp.int32),
            pltpu.VMEM((2, 1, _DB, B), jnp.float32),
            pltpu.SemaphoreType.DMA,
            pltpu.SemaphoreType.DMA,
            pltpu.SemaphoreType.DMA,
        ],
        compiler_params=pltpu.CompilerParams(needs_layout_passes=False),
    )
    def k(wt_hbm, xt_hbm, out_hbm, wt_v, idx_v, out_v, sem_t, sem_i, sem_o):
        wid = lax.axis_index("s") * _NC + lax.axis_index("c")
        u0 = wid * base + jnp.minimum(wid, rem)
        u1 = u0 + base + jnp.where(wid < rem, 1, 0)

        def start_idx(u, p):
            pltpu.async_copy(xt_hbm.at[pl.ds(u // nd, 1)], idx_v.at[p], sem_i)

        def wait_idx():
            pltpu.make_async_copy(xt_hbm.at[pl.ds(0, 1)], idx_v.at[0], sem_i).wait()

        def start_out(u, p):
            dt = u % nd
            pltpu.async_copy(
                out_v.at[p],
                out_hbm.at[
                    pl.ds(u // nd, 1),
                    pl.ds(pl.multiple_of(dt * _DB, _DB), _DB),
                ],
                sem_o,
            )

        def wait_out():
            pltpu.make_async_copy(
                out_v.at[0], out_hbm.at[pl.ds(0, 1), pl.ds(0, _DB)], sem_o
            ).wait()

        tbl = pltpu.async_copy(wt_hbm, wt_v, sem_t)
        start_idx(u0, 0)
        tbl.wait()

        def unit(u, _):
            p = lax.rem(u - u0, 2)
            dt = u % nd

            @pl.when(u + 1 < u1)
            def _():
                start_idx(u + 1, 1 - p)

            wait_idx()

            @pl.when(u - u0 >= 2)
            def _():
                wait_out()

            bases = [
                jnp.full((16,), (dt * _DB + d8) * _VP, jnp.int32)
                for d8 in range(_DB)
            ]

            def col(g, _):
                iv = idx_v[p, 0, pl.ds(g * 16, 16)]
                vals = [
                    plsc.load_gather(wt_v, [iv + bases[d8]])
                    for d8 in range(_DB)
                ]
                for d8 in range(_DB):
                    out_v[p, 0, d8, pl.ds(g * 16, 16)] = vals[d8]
                return ()

            lax.fori_loop(0, B // 16, col, (), unroll=2)
            start_out(u, p)
            return ()

        lax.fori_loop(u0, u1, unit, ())
        wait_out()
        wait_out()

    return k(Wt, xT)


def kernel(x, W):
    B, L = x.shape
    V, D = W.shape
    Wt = jnp.pad(W.T, ((0, 0), (0, _VP - V))).reshape(-1)
    out_t = _embed_t(Wt, x.T, B, L, D)
    return jnp.transpose(out_t, (2, 0, 1))


# speedup vs baseline: 2.0492x; 1.1662x over previous
import functools

import jax
import jax.numpy as jnp
from jax import lax
from jax.experimental import pallas as pl
from jax.experimental.pallas import tpu as pltpu
from jax.experimental.pallas import tpu_sc as plsc

_info = plsc.get_sparse_core_info()
_NC, _NS = _info.num_cores, _info.num_subcores
_NW = _NC * _NS

_DB = 8
_VP = 1024


@functools.partial(jax.jit, static_argnums=(2, 3, 4))
def _embed_t(Wt, xT, B, L, d):
    nd = d // _DB
    ng = _NW // nd
    base = L // ng
    rem = L % ng
    mesh = plsc.VectorSubcoreMesh(core_axis_name="c", subcore_axis_name="s")
    assert base >= 2

    @functools.partial(
        pl.kernel,
        mesh=mesh,
        out_type=jax.ShapeDtypeStruct((L, d, B), jnp.float32),
        scratch_types=[
            pltpu.VMEM((_DB * _VP,), jnp.float32),
            pltpu.VMEM((2, 1, B), jnp.int32),
            pltpu.VMEM((2, 1, _DB, B), jnp.float32),
            pltpu.SemaphoreType.DMA,
            pltpu.SemaphoreType.DMA,
            pltpu.SemaphoreType.DMA,
        ],
        compiler_params=pltpu.CompilerParams(needs_layout_passes=False),
    )
    def k(wt_hbm, xt_hbm, out_hbm, wt_v, idx_v, out_v, sem_t, sem_i, sem_o):
        wid = lax.axis_index("s") * _NC + lax.axis_index("c")
        dt = lax.rem(wid, nd)
        g = wid // nd
        l0 = g * base + jnp.minimum(g, rem)
        l1 = l0 + base + jnp.where(g < rem, 1, 0)

        bases = [jnp.full((16,), d8 * _VP, jnp.int32) for d8 in range(_DB)]

        def start_idx(l, p):
            pltpu.async_copy(xt_hbm.at[pl.ds(l, 1)], idx_v.at[p], sem_i)

        def wait_idx():
            pltpu.make_async_copy(xt_hbm.at[pl.ds(0, 1)], idx_v.at[0], sem_i).wait()

        def start_out(l, p):
            pltpu.async_copy(
                out_v.at[p],
                out_hbm.at[
                    pl.ds(l, 1),
                    pl.ds(pl.multiple_of(dt * _DB, _DB), _DB),
                ],
                sem_o,
            )

        def wait_out():
            pltpu.make_async_copy(
                out_v.at[0], out_hbm.at[pl.ds(0, 1), pl.ds(0, _DB)], sem_o
            ).wait()

        tbl = pltpu.async_copy(
            wt_hbm.at[pl.ds(pl.multiple_of(dt * _DB * _VP, 8), _DB * _VP)],
            wt_v,
            sem_t,
        )
        start_idx(l0, 0)
        tbl.wait()

        def unit(l, _):
            p = lax.rem(l - l0, 2)

            @pl.when(l + 1 < l1)
            def _():
                start_idx(l + 1, 1 - p)

            wait_idx()

            @pl.when(l - l0 >= 2)
            def _():
                wait_out()

            def col(cg, _):
                iv = idx_v[p, 0, pl.ds(cg * 16, 16)]
                vals = [
                    plsc.load_gather(wt_v, [iv + bases[d8]])
                    for d8 in range(_DB)
                ]
                for d8 in range(_DB):
                    out_v[p, 0, d8, pl.ds(cg * 16, 16)] = vals[d8]
                return ()

            lax.fori_loop(0, B // 16, col, (), unroll=4)
            start_out(l, p)
            return ()

        lax.fori_loop(l0, l1, unit, ())
        wait_out()
        wait_out()

    return k(Wt, xT)


def kernel(x, W):
    B, L = x.shape
    V, D = W.shape
    Wt = jnp.pad(W.T, ((0, 0), (0, _VP - V))).reshape(-1)
    out_t = _embed_t(Wt, x.T, B, L, D)
    return jnp.transpose(out_t, (2, 0, 1))


# speedup vs baseline: 2.1975x; 1.0724x over previous
import functools

import jax
import jax.numpy as jnp
from jax import lax
from jax.experimental import pallas as pl
from jax.experimental.pallas import tpu as pltpu
from jax.experimental.pallas import tpu_sc as plsc

_info = plsc.get_sparse_core_info()
_NC, _NS = _info.num_cores, _info.num_subcores
_NW = _NC * _NS

_DB = 8
_VP = 1024


@functools.partial(jax.jit, static_argnums=(2, 3, 4))
def _embed_t(Wt, xT, B, L, d):
    nd = d // _DB
    ng = _NW // nd
    base = L // ng
    rem = L % ng
    mesh = plsc.VectorSubcoreMesh(core_axis_name="c", subcore_axis_name="s")
    assert base >= 2

    @functools.partial(
        pl.kernel,
        mesh=mesh,
        out_type=jax.ShapeDtypeStruct((L, d, B), jnp.float32),
        scratch_types=[
            pltpu.VMEM((_DB * _VP,), jnp.float32),
            pltpu.VMEM((2, 1, B + 16), jnp.int32),
            pltpu.VMEM((2, 1, _DB, B), jnp.float32),
            pltpu.SemaphoreType.DMA,
            pltpu.SemaphoreType.DMA,
            pltpu.SemaphoreType.DMA,
        ],
        compiler_params=pltpu.CompilerParams(needs_layout_passes=False),
    )
    def k(wt_hbm, xt_hbm, out_hbm, wt_v, idx_v, out_v, sem_t, sem_i, sem_o):
        wid = lax.axis_index("s") * _NC + lax.axis_index("c")
        dt = lax.rem(wid, nd)
        g = wid // nd
        l0 = g * base + jnp.minimum(g, rem)
        l1 = l0 + base + jnp.where(g < rem, 1, 0)

        bases = [jnp.full((16,), d8 * _VP, jnp.int32) for d8 in range(_DB)]

        def start_idx(l, p):
            pltpu.async_copy(
                xt_hbm.at[pl.ds(l, 1)], idx_v.at[p, :, pl.ds(0, B)], sem_i
            )

        def wait_idx():
            pltpu.make_async_copy(
                xt_hbm.at[pl.ds(0, 1)], idx_v.at[0, :, pl.ds(0, B)], sem_i
            ).wait()

        def start_out(l, p):
            pltpu.async_copy(
                out_v.at[p],
                out_hbm.at[
                    pl.ds(l, 1),
                    pl.ds(pl.multiple_of(dt * _DB, _DB), _DB),
                ],
                sem_o,
            )

        def wait_out():
            pltpu.make_async_copy(
                out_v.at[0], out_hbm.at[pl.ds(0, 1), pl.ds(0, _DB)], sem_o
            ).wait()

        tbl = pltpu.async_copy(
            wt_hbm.at[pl.ds(pl.multiple_of(dt * _DB * _VP, 8), _DB * _VP)],
            wt_v,
            sem_t,
        )
        start_idx(l0, 0)
        tbl.wait()

        def unit(l, _):
            p = lax.rem(l - l0, 2)

            @pl.when(l + 1 < l1)
            def _():
                start_idx(l + 1, 1 - p)

            wait_idx()

            @pl.when(l - l0 >= 2)
            def _():
                wait_out()

            def col(cg, iv):
                nxt = idx_v[p, 0, pl.ds(cg * 16 + 16, 16)]
                vals = [
                    plsc.load_gather(wt_v, [iv + bases[d8]])
                    for d8 in range(_DB)
                ]
                for d8 in range(_DB):
                    out_v[p, 0, d8, pl.ds(cg * 16, 16)] = vals[d8]
                return nxt

            iv0 = idx_v[p, 0, pl.ds(0, 16)]
            lax.fori_loop(0, B // 16, col, iv0, unroll=4)
            start_out(l, p)
            return ()

        lax.fori_loop(l0, l1, unit, ())
        wait_out()
        wait_out()

    return k(Wt, xT)


def kernel(x, W):
    B, L = x.shape
    V, D = W.shape
    Wt = jnp.pad(W.T, ((0, 0), (0, _VP - V))).reshape(-1)
    out_t = _embed_t(Wt, x.T, B, L, D)
    return jnp.transpose(out_t, (2, 0, 1))


# speedup vs baseline: 2.2241x; 1.0121x over previous
import functools

import jax
import jax.numpy as jnp
from jax import lax
from jax.experimental import pallas as pl
from jax.experimental.pallas import tpu as pltpu
from jax.experimental.pallas import tpu_sc as plsc

_info = plsc.get_sparse_core_info()
_NC, _NS = _info.num_cores, _info.num_subcores
_NW = _NC * _NS

_DB = 8
_VP = 1024


@functools.partial(jax.jit, static_argnums=(2, 3, 4))
def _embed_t(Wt, xT, B, L, d):
    nd = d // _DB
    ng = _NW // nd
    base = L // ng
    rem = L % ng
    mesh = plsc.VectorSubcoreMesh(core_axis_name="c", subcore_axis_name="s")
    assert base >= 2

    @functools.partial(
        pl.kernel,
        mesh=mesh,
        out_type=jax.ShapeDtypeStruct((L, d, B), jnp.float32),
        scratch_types=[
            pltpu.VMEM((_DB * _VP,), jnp.float32),
            pltpu.VMEM((2, 1, B + 16), jnp.int32),
            pltpu.VMEM((2, 1, _DB, B), jnp.float32),
            pltpu.SemaphoreType.DMA,
            pltpu.SemaphoreType.DMA,
            pltpu.SemaphoreType.DMA,
        ],
        compiler_params=pltpu.CompilerParams(needs_layout_passes=False),
    )
    def k(wt_hbm, xt_hbm, out_hbm, wt_v, idx_v, out_v, sem_t, sem_i, sem_o):
        wid = lax.axis_index("s") * _NC + lax.axis_index("c")
        dt = lax.rem(wid, nd)
        g = wid // nd
        l0 = g * base + jnp.minimum(g, rem)
        l1 = l0 + base + jnp.where(g < rem, 1, 0)

        bases = [jnp.full((16,), d8 * _VP, jnp.int32) for d8 in range(_DB)]

        def start_idx(l, p):
            pltpu.async_copy(
                xt_hbm.at[pl.ds(l, 1)], idx_v.at[p, :, pl.ds(0, B)], sem_i
            )

        def wait_idx():
            pltpu.make_async_copy(
                xt_hbm.at[pl.ds(0, 1)], idx_v.at[0, :, pl.ds(0, B)], sem_i
            ).wait()

        def start_out(l, p):
            pltpu.async_copy(
                out_v.at[p],
                out_hbm.at[
                    pl.ds(l, 1),
                    pl.ds(pl.multiple_of(dt * _DB, _DB), _DB),
                ],
                sem_o,
            )

        def wait_out():
            pltpu.make_async_copy(
                out_v.at[0], out_hbm.at[pl.ds(0, 1), pl.ds(0, _DB)], sem_o
            ).wait()

        tbl = pltpu.async_copy(
            wt_hbm.at[pl.ds(pl.multiple_of(dt * _DB * _VP, 8), _DB * _VP)],
            wt_v,
            sem_t,
        )
        start_idx(l0, 0)
        tbl.wait()

        def unit(l, _):
            p = lax.rem(l - l0, 2)

            @pl.when(l + 1 < l1)
            def _():
                start_idx(l + 1, 1 - p)

            wait_idx()

            @pl.when(l - l0 >= 2)
            def _():
                wait_out()

            def col(cg, iv):
                nxt = idx_v[p, 0, pl.ds(cg * 16 + 16, 16)]
                vals = [
                    plsc.load_gather(wt_v, [iv + bases[d8]])
                    for d8 in range(_DB)
                ]
                for d8 in range(_DB):
                    out_v[p, 0, d8, pl.ds(cg * 16, 16)] = vals[d8]
                return nxt

            iv0 = idx_v[p, 0, pl.ds(0, 16)]
            lax.fori_loop(0, B // 16, col, iv0, unroll=8)
            start_out(l, p)
            return ()

        lax.fori_loop(l0, l1, unit, ())
        wait_out()
        wait_out()

    return k(Wt, xT)


def kernel(x, W):
    B, L = x.shape
    V, D = W.shape
    Wt = jnp.pad(W.T, ((0, 0), (0, _VP - V))).reshape(-1)
    out_t = _embed_t(Wt, x.T, B, L, D)
    return jnp.transpose(out_t, (2, 0, 1))


# speedup vs baseline: 2.2713x; 1.0212x over previous
import functools

import jax
import jax.numpy as jnp
from jax import lax
from jax.experimental import pallas as pl
from jax.experimental.pallas import tpu as pltpu
from jax.experimental.pallas import tpu_sc as plsc

_info = plsc.get_sparse_core_info()
_NC, _NS = _info.num_cores, _info.num_subcores
_NW = _NC * _NS

_DB = 8
_VP = 1024


@functools.partial(jax.jit, static_argnums=(2, 3, 4))
def _embed_t(Wt, xT, B, L, d):
    nd = d // _DB
    ng = _NW // nd
    base = L // ng
    rem = L % ng
    mesh = plsc.VectorSubcoreMesh(core_axis_name="c", subcore_axis_name="s")
    assert base >= 2

    @functools.partial(
        pl.kernel,
        mesh=mesh,
        out_type=jax.ShapeDtypeStruct((L, d, B), jnp.float32),
        scratch_types=[
            pltpu.VMEM((_DB * _VP,), jnp.float32),
            pltpu.VMEM((2, 1, B + 16), jnp.int32),
            pltpu.VMEM((2, 1, _DB, B), jnp.float32),
            pltpu.SemaphoreType.DMA,
            pltpu.SemaphoreType.DMA,
            pltpu.SemaphoreType.DMA,
        ],
        compiler_params=pltpu.CompilerParams(needs_layout_passes=False),
    )
    def k(wt_hbm, xt_hbm, out_hbm, wt_v, idx_v, out_v, sem_t, sem_i, sem_o):
        wid = lax.axis_index("s") * _NC + lax.axis_index("c")
        dt = lax.rem(wid, nd)
        g = wid // nd
        l0 = g * base + jnp.minimum(g, rem)
        l1 = l0 + base + jnp.where(g < rem, 1, 0)

        bases = [jnp.full((16,), d8 * _VP, jnp.int32) for d8 in range(_DB)]

        def start_idx(l, p):
            pltpu.async_copy(
                xt_hbm.at[pl.ds(l, 1)], idx_v.at[p, :, pl.ds(0, B)], sem_i
            )

        def wait_idx():
            pltpu.make_async_copy(
                xt_hbm.at[pl.ds(0, 1)], idx_v.at[0, :, pl.ds(0, B)], sem_i
            ).wait()

        def start_out(l, p):
            pltpu.async_copy(
                out_v.at[p],
                out_hbm.at[
                    pl.ds(l, 1),
                    pl.ds(pl.multiple_of(dt * _DB, _DB), _DB),
                ],
                sem_o,
            )

        def wait_out():
            pltpu.make_async_copy(
                out_v.at[0], out_hbm.at[pl.ds(0, 1), pl.ds(0, _DB)], sem_o
            ).wait()

        tbl = pltpu.async_copy(
            wt_hbm.at[pl.ds(pl.multiple_of(dt * _DB * _VP, 8), _DB * _VP)],
            wt_v,
            sem_t,
        )
        start_idx(l0, 0)
        tbl.wait()

        def unit(l, _):
            p = lax.rem(l - l0, 2)

            @pl.when(l + 1 < l1)
            def _():
                start_idx(l + 1, 1 - p)

            wait_idx()

            @pl.when(l - l0 >= 2)
            def _():
                wait_out()

            def col(cg, carry):
                iv, vals = carry
                nxt = idx_v[p, 0, pl.ds(cg * 16 + 16, 16)]
                new = tuple(
                    plsc.load_gather(wt_v, [iv + bases[d8]])
                    for d8 in range(_DB)
                )
                for d8 in range(_DB):
                    out_v[p, 0, d8, pl.ds(cg * 16 - 16, 16)] = vals[d8]
                return nxt, new

            iv0 = idx_v[p, 0, pl.ds(0, 16)]
            vals0 = tuple(
                plsc.load_gather(wt_v, [iv0 + bases[d8]]) for d8 in range(_DB)
            )
            iv1 = idx_v[p, 0, pl.ds(16, 16)]
            _, last = lax.fori_loop(
                1, B // 16, col, (iv1, vals0), unroll=8
            )
            for d8 in range(_DB):
                out_v[p, 0, d8, pl.ds(B - 16, 16)] = last[d8]
            start_out(l, p)
            return ()

        lax.fori_loop(l0, l1, unit, ())
        wait_out()
        wait_out()

    return k(Wt, xT)


def kernel(x, W):
    B, L = x.shape
    V, D = W.shape
    Wt = jnp.pad(W.T, ((0, 0), (0, _VP - V))).reshape(-1)
    out_t = _embed_t(Wt, x.T, B, L, D)
    return jnp.transpose(out_t, (2, 0, 1))
